# Initial kernel scaffold; baseline (speedup 1.0000x reference)
import jax
import jax.numpy as jnp
from jax.experimental import pallas as pl


def kernel(act, location, duration, tst, tet, travel, edge_index, batch, node_emb0, node_emb1, edge_emb0, W, att_src, att_dst, W_edge, att_edge, gat_bias, fc_W, fc_b):
    raise NotImplementedError("write your pallas kernel here")


# speedup vs baseline: 26.2331x; 26.2331x over previous
import functools

import jax
import jax.numpy as jnp
from jax import lax
from jax.experimental import pallas as pl
from jax.experimental.pallas import tpu as pltpu
from jax.experimental.pallas import tpu_sc as plsc

N = 100000
E = 1600000
G = 256
H = 32
HH = 16
NCORE = 2
NSUB = 16
NW = NCORE * NSUB

NP = 102400
PER_W_N = NP // NW
GC = 640
PER_S_E = E // NSUB
EB = 800
STRIPE = NP // NSUB
SCP = 800

_mesh = plsc.VectorSubcoreMesh(core_axis_name="c", subcore_axis_name="s")
_SC_PARAMS = pltpu.CompilerParams(use_tc_tiling_on_sc=False,
                                  needs_layout_passes=False)

F32 = jnp.float32
I32 = jnp.int32


def _emb_body(act_hbm, loc_hbm, e0_hbm, e1_hbm, x_hbm, ia, il, r0, r1, s0, s1):
    c = lax.axis_index("c")
    s = lax.axis_index("s")
    wid = s * NCORE + c
    base = wid * PER_W_N

    @pl.loop(0, PER_W_N, step=GC)
    def _(off):
        b = base + off
        pltpu.sync_copy(act_hbm.at[pl.ds(b, GC)], ia)
        pltpu.sync_copy(loc_hbm.at[pl.ds(b, GC)], il)
        cp0 = pltpu.async_copy(e0_hbm.at[ia], r0, s0)
        cp1 = pltpu.async_copy(e1_hbm.at[il], r1, s1)
        cp0.wait()
        cp1.wait()

        @pl.loop(0, GC)
        def _(i):
            for j in range(2):
                sl = (i, pl.ds(j * 16, 16))
                r0[sl] = jnp.maximum(r0[sl] + r1[sl], 0.0)

        pltpu.sync_copy(r0, x_hbm.at[pl.ds(b, GC)])


def _emb_lookup(act_p, loc_p, e0, e1):
    kern = pl.kernel(
        _emb_body,
        out_type=jax.ShapeDtypeStruct((NP, H), F32),
        mesh=_mesh,
        scratch_types=[
            pltpu.VMEM((GC,), I32),
            pltpu.VMEM((GC,), I32),
            pltpu.VMEM((GC, H), F32),
            pltpu.VMEM((GC, H), F32),
            pltpu.SemaphoreType.DMA,
            pltpu.SemaphoreType.DMA,
        ],
        compiler_params=_SC_PARAMS,
    )
    return kern(act_p, loc_p, e0, e1)


BLK = 2048
NB = NP // BLK

_HIGH = lax.Precision.HIGHEST


def _node_body(x_ref, w_ref, asr_ref, adr_ref, e0_ref, we_ref, ae_ref,
               hlo_ref, hhi_ref, as_ref, ad_ref, econ_ref):
    i = pl.program_id(0)
    x = x_ref[...]
    h = lax.dot_general(x, w_ref[...], (((1,), (0,)), ((), ())), precision=_HIGH)
    hlo_ref[...] = h[:, :HH]
    hhi_ref[...] = h[:, HH:]
    as_ref[...] = jnp.sum(h * asr_ref[...], axis=1, keepdims=True)
    ad_ref[...] = jnp.sum(h * adr_ref[...], axis=1, keepdims=True)

    @pl.when(i == 0)
    def _():
        we = lax.dot_general(ae_ref[...], we_ref[...], (((1,), (1,)), ((), ())),
                             precision=_HIGH)
        te = lax.dot_general(we[:, :H], e0_ref[...], (((1,), (1,)), ((), ())),
                             precision=_HIGH)
        te16 = jnp.concatenate([te, jnp.zeros((1, 6), F32)], axis=1)
        w0 = jnp.broadcast_to(we[:, H:H + 1], (1, 16))
        w1 = jnp.broadcast_to(we[:, H + 1:H + 2], (1, 16))
        w2 = jnp.broadcast_to(we[:, H + 2:H + 3], (1, 16))
        econ_ref[...] = jnp.concatenate(
            [te16, w0, w1, w2, jnp.zeros((4, 16), F32)], axis=0)


def _node_stage(x, W, att_src2, att_dst2, edge_emb0, W_edge, att_edge2):
    return pl.pallas_call(
        _node_body,
        grid=(NB,),
        in_specs=[
            pl.BlockSpec((BLK, H), lambda i: (i, 0)),
            pl.BlockSpec((H, H), lambda i: (0, 0)),
            pl.BlockSpec((1, H), lambda i: (0, 0)),
            pl.BlockSpec((1, H), lambda i: (0, 0)),
            pl.BlockSpec((10, H), lambda i: (0, 0)),
            pl.BlockSpec((H + 3, H), lambda i: (0, 0)),
            pl.BlockSpec((1, H), lambda i: (0, 0)),
        ],
        out_specs=[
            pl.BlockSpec((BLK, HH), lambda i: (i, 0)),
            pl.BlockSpec((BLK, HH), lambda i: (i, 0)),
            pl.BlockSpec((BLK, 1), lambda i: (i, 0)),
            pl.BlockSpec((BLK, 1), lambda i: (i, 0)),
            pl.BlockSpec((8, 16), lambda i: (0, 0)),
        ],
        out_shape=[
            jax.ShapeDtypeStruct((NP, HH), F32),
            jax.ShapeDtypeStruct((NP, HH), F32),
            jax.ShapeDtypeStruct((NP, 1), F32),
            jax.ShapeDtypeStruct((NP, 1), F32),
            jax.ShapeDtypeStruct((8, 16), F32),
        ],
    )(x, W, att_src2, att_dst2, edge_emb0, W_edge, att_edge2)


def _zero_phase(s, rows, exv, acc_sh, den_sh, do_den):
    @pl.loop(0, EB)
    def _(i):
        rows[i, :] = jnp.zeros((16,), F32)

    @pl.loop(0, EB, step=16)
    def _(i):
        exv[pl.ds(i, 16)] = jnp.zeros((16,), F32)

    srow = s * STRIPE
    for k in range(STRIPE // SCP):
        pltpu.sync_copy(rows.at[pl.ds(0, SCP)],
                        acc_sh.at[pl.ds(srow + k * SCP, SCP)])
    if do_den:
        for k in range(STRIPE // SCP):
            pltpu.sync_copy(exv.at[pl.ds(0, SCP)],
                            den_sh.at[pl.ds(srow + k * SCP, SCP)])


def _sweep_phase(s, src_hbm, dst_hbm, trav_hbm, du_hbm, ts_hbm, tt_hbm,
                 as_hbm, ad_hbm, h_hbm, econv,
                 srcv, dstv, travv, duv, tsv, ttv, asv, adv, exv, rows,
                 sema, semb, semr, acc_sh, den_sh, do_den):
    zero16 = jnp.zeros((16,), I32)
    w0v = plsc.load_gather(econv, [jnp.full((16,), 1, I32), zero16])
    w1v = plsc.load_gather(econv, [jnp.full((16,), 2, I32), zero16])
    w2v = plsc.load_gather(econv, [jnp.full((16,), 3, I32), zero16])
    ebase = s * PER_S_E

    @pl.loop(0, PER_S_E, step=EB)
    def _(off):
        b = ebase + off
        pltpu.sync_copy(src_hbm.at[pl.ds(b, EB)], srcv)
        pltpu.sync_copy(dst_hbm.at[pl.ds(b, EB)], dstv)
        pltpu.sync_copy(trav_hbm.at[pl.ds(b, EB)], travv)
        pltpu.sync_copy(du_hbm.at[pl.ds(b, EB)], duv)
        pltpu.sync_copy(ts_hbm.at[pl.ds(b, EB)], tsv)
        pltpu.sync_copy(tt_hbm.at[pl.ds(b, EB)], ttv)
        cpa = pltpu.async_copy(as_hbm.at[srcv], asv, sema)
        cpb = pltpu.async_copy(ad_hbm.at[dstv], adv, semb)
        cpr = pltpu.async_copy(h_hbm.at[srcv], rows, semr)
        cpa.wait()
        cpb.wait()
        cpr.wait()

        @pl.loop(0, EB, step=16)
        def _(g):
            tr = travv[pl.ds(g, 16)]
            aev = plsc.load_gather(econv, [zero16, tr])
            al = (asv[pl.ds(g, 16)] + adv[pl.ds(g, 16)] + aev
                  + duv[pl.ds(g, 16)] * w0v
                  + tsv[pl.ds(g, 16)] * w1v
                  + ttv[pl.ds(g, 16)] * w2v)
            al = jnp.maximum(al, 0.2 * al)
            ex = jnp.exp(al)
            exv[pl.ds(g, 16)] = ex
            for j in range(16):
                spl = plsc.load_gather(exv, [jnp.broadcast_to(g + j, (16,))])
                rows[g + j, :] = rows[g + j, :] * spl

        pltpu.sync_copy(rows, acc_sh.at[dstv], add=True)
        if do_den:
            pltpu.sync_copy(exv, den_sh.at[dstv], add=True)


def _out_phase(s, acc_sh, den_sh, acc_hbm, den_hbm, do_den):
    srow = s * STRIPE
    for k in range(STRIPE // SCP):
        sl = pl.ds(srow + k * SCP, SCP)
        pltpu.sync_copy(acc_sh.at[sl], acc_hbm.at[sl])
    if do_den:
        for k in range(STRIPE // SCP):
            sl = pl.ds(srow + k * SCP, SCP)
            pltpu.sync_copy(den_sh.at[sl], den_hbm.at[sl])


def _edge_body(src_hbm, dst_hbm, trav_hbm, du_hbm, ts_hbm, tt_hbm,
               as_hbm, ad_hbm, hlo_hbm, hhi_hbm, econ_hbm,
               acclo_hbm, acchi_hbm, den_hbm,
               srcv, dstv, travv, duv, tsv, ttv, asv, adv, exv, rows, econv,
               sema, semb, semr, acc_sh, den_sh):
    c = lax.axis_index("c")
    s = lax.axis_index("s")
    pltpu.sync_copy(econ_hbm, econv)

    @pl.when(c == 0)
    def _():
        _zero_phase(s, rows, exv, acc_sh, den_sh, True)

    @pl.when(c == 1)
    def _():
        _zero_phase(s, rows, exv, acc_sh, den_sh, False)

    plsc.subcore_barrier()

    common = (srcv, dstv, travv, duv, tsv, ttv, asv, adv, exv, rows,
              sema, semb, semr, acc_sh, den_sh)

    @pl.when(c == 0)
    def _():
        _sweep_phase(s, src_hbm, dst_hbm, trav_hbm, du_hbm, ts_hbm, tt_hbm,
                     as_hbm, ad_hbm, hlo_hbm, econv, *common, True)

    @pl.when(c == 1)
    def _():
        _sweep_phase(s, src_hbm, dst_hbm, trav_hbm, du_hbm, ts_hbm, tt_hbm,
                     as_hbm, ad_hbm, hhi_hbm, econv, *common, False)

    plsc.subcore_barrier()

    @pl.when(c == 0)
    def _():
        _out_phase(s, acc_sh, den_sh, acclo_hbm, den_hbm, True)

    @pl.when(c == 1)
    def _():
        _out_phase(s, acc_sh, den_sh, acchi_hbm, den_hbm, False)


def _edge_stage(src, dst, trav, du, ts, tt, as1, ad1, hlo, hhi, econ):
    kern = pl.kernel(
        _edge_body,
        out_type=(
            jax.ShapeDtypeStruct((NP, HH), F32),
            jax.ShapeDtypeStruct((NP, HH), F32),
            jax.ShapeDtypeStruct((NP,), F32),
        ),
        mesh=_mesh,
        scratch_types=[
            pltpu.VMEM((EB,), I32),
            pltpu.VMEM((EB,), I32),
            pltpu.VMEM((EB,), I32),
            pltpu.VMEM((EB,), F32),
            pltpu.VMEM((EB,), F32),
            pltpu.VMEM((EB,), F32),
            pltpu.VMEM((EB,), F32),
            pltpu.VMEM((EB,), F32),
            pltpu.VMEM((EB,), F32),
            pltpu.VMEM((EB, 16), F32),
            pltpu.VMEM((8, 16), F32),
            pltpu.SemaphoreType.DMA,
            pltpu.SemaphoreType.DMA,
            pltpu.SemaphoreType.DMA,
            pltpu.VMEM_SHARED((NP, HH), F32),
            pltpu.VMEM_SHARED((NP,), F32),
        ],
        compiler_params=_SC_PARAMS,
    )
    return kern(src, dst, trav, du, ts, tt, as1, ad1, hlo, hhi, econ)


BLK4 = 2048
NB4 = NP // BLK4


def _pool_body(acclo_ref, acchi_ref, den_ref, batch_ref, bias_ref,
               fcw_ref, fcb_ref, out_ref, gsum, cnt):
    i = pl.program_id(0)

    @pl.when(i == 0)
    def _():
        gsum[...] = jnp.zeros_like(gsum)
        cnt[...] = jnp.zeros_like(cnt)

    a = jnp.concatenate([acclo_ref[...], acchi_ref[...]], axis=1)
    den = den_ref[...]
    outn = jnp.maximum(a / (den + 1e-16) + bias_ref[...], 0.0)
    bt = batch_ref[...]
    oh = (bt == lax.broadcasted_iota(I32, (BLK4, G), 1)).astype(F32)
    gsum[...] += lax.dot_general(oh, outn, (((0,), (0,)), ((), ())),
                                 precision=_HIGH)
    cnt[...] += lax.dot_general(oh, jnp.ones((BLK4, 1), F32),
                                (((0,), (0,)), ((), ())), precision=_HIGH)

    @pl.when(i == NB4 - 1)
    def _():
        gmean = gsum[...] / jnp.maximum(cnt[...], 1.0)
        logits = lax.dot_general(gmean, fcw_ref[...], (((1,), (0,)), ((), ())),
                                 precision=_HIGH) + fcb_ref[...]
        m = jnp.max(logits, axis=1, keepdims=True)
        sh = logits - m
        out_ref[...] = sh - jnp.log(jnp.sum(jnp.exp(sh), axis=1, keepdims=True))


def _pool_stage(acclo, acchi, den2, batch2, bias2, fc_W, fcb2):
    return pl.pallas_call(
        _pool_body,
        grid=(NB4,),
        in_specs=[
            pl.BlockSpec((BLK4, HH), lambda i: (i, 0)),
            pl.BlockSpec((BLK4, HH), lambda i: (i, 0)),
            pl.BlockSpec((BLK4, 1), lambda i: (i, 0)),
            pl.BlockSpec((BLK4, 1), lambda i: (i, 0)),
            pl.BlockSpec((1, H), lambda i: (0, 0)),
            pl.BlockSpec((H, 16), lambda i: (0, 0)),
            pl.BlockSpec((1, 16), lambda i: (0, 0)),
        ],
        out_specs=pl.BlockSpec((G, 16), lambda i: (0, 0)),
        out_shape=jax.ShapeDtypeStruct((G, 16), F32),
        scratch_shapes=[
            pltpu.VMEM((G, H), F32),
            pltpu.VMEM((G, 1), F32),
        ],
    )(acclo, acchi, den2, batch2, bias2, fc_W, fcb2)


def kernel(act, location, duration, tst, tet, travel, edge_index, batch,
           node_emb0, node_emb1, edge_emb0, W, att_src, att_dst,
           W_edge, att_edge, gat_bias, fc_W, fc_b):
    pad_n = NP - N
    act_p = jnp.concatenate([act.astype(I32), jnp.zeros((pad_n,), I32)])
    loc_p = jnp.concatenate([location.astype(I32), jnp.zeros((pad_n,), I32)])
    batch_p = jnp.concatenate(
        [batch.astype(I32), jnp.full((pad_n,), G, I32)]).reshape(NP, 1)
    src = edge_index[0].astype(I32)
    dst = edge_index[1].astype(I32)
    trav = travel.astype(I32)

    x = _emb_lookup(act_p, loc_p, node_emb0, node_emb1)
    hlo, hhi, as2, ad2, econ = _node_stage(
        x, W, att_src.reshape(1, H), att_dst.reshape(1, H),
        edge_emb0, W_edge, att_edge.reshape(1, H))
    as1 = as2.reshape(NP)
    ad1 = ad2.reshape(NP)
    acclo, acchi, den = _edge_stage(
        src, dst, trav, duration, tst, tet, as1, ad1, hlo, hhi, econ)
    return _pool_stage(acclo, acchi, den.reshape(NP, 1), batch_p,
                       gat_bias.reshape(1, H), fc_W, fc_b.reshape(1, 16))
